# Initial kernel scaffold; baseline (speedup 1.0000x reference)
import jax
import jax.numpy as jnp
from jax.experimental import pallas as pl


def kernel(subword_ids, mask_indices, mask_values, emb, ln_weight, ln_bias):
    raise NotImplementedError("write your pallas kernel here")


# speedup vs baseline: 1.0335x; 1.0335x over previous
import jax
import jax.numpy as jnp
from jax import lax
from jax.experimental import pallas as pl
from jax.experimental.pallas import tpu as pltpu
from jax.experimental.pallas import tpu_sc as plsc

B, N, S, V, H = 16, 256, 512, 30522, 768
NNZ = 8192
EPS = 1e-12

NC, NS, L = 2, 16, 16
NW = NC * NS
SEGS = B * N
SPT = SEGS // NW
SEG_SHIFT = 7
EC = 1024
NCHUNK = NNZ // EC
GC = 16
HCH = H // L
MAGIC = 0x5F3759DF

_GDN = lax.GatherDimensionNumbers(
    offset_dims=(), collapsed_slice_dims=(0,), start_index_map=(0,))


def _xlane(x, idx):
    return lax.gather(x, idx[:, None], _GDN, (1,),
                      mode=lax.GatherScatterMode.PROMISE_IN_BOUNDS)


def _lanesum(x):
    iota = lax.iota(jnp.int32, L)
    for s in (1, 2, 4, 8):
        x = x + _xlane(x, iota ^ s)
    return x


def _prefix_sum(x):
    iota = lax.iota(jnp.int32, L)
    zero = jnp.zeros((L,), x.dtype)
    for s in (1, 2, 4, 8):
        sh = _xlane(x, jnp.maximum(iota - s, 0))
        x = x + jnp.where(iota >= s, sh, zero)
    return x


def _body(sub_hbm, b_hbm, n_hbm, s_hbm, v_hbm, emb_hbm, lnw_hbm, lnb_hbm,
          out_hbm, acc, sub_v, eb, en, es, ev, cvoc, cseg, cval, rowbuf,
          lnw_v, lnb_v, sem0):
    cid = lax.axis_index("c")
    sid = lax.axis_index("s")
    wid = sid * NC + cid
    widv = jnp.broadcast_to(wid.astype(jnp.int32), (L,))

    zero16 = jnp.zeros((L,), jnp.float32)
    izero16 = jnp.zeros((L,), jnp.int32)

    pltpu.sync_copy(sub_hbm, sub_v)
    pltpu.sync_copy(lnw_hbm, lnw_v)
    pltpu.sync_copy(lnb_hbm, lnb_v)

    def zacc(r, _):
        def zcol(cb, _):
            for u in range(8):
                acc[r, pl.ds(cb * 128 + u * 16, L)] = zero16
            return 0
        lax.fori_loop(0, 6, zcol, 0)
        return 0
    lax.fori_loop(0, SPT, zacc, 0)

    def chunk(k, _):
        off = k * EC
        pltpu.sync_copy(b_hbm.at[pl.ds(off, EC)], eb)
        pltpu.sync_copy(n_hbm.at[pl.ds(off, EC)], en)
        pltpu.sync_copy(s_hbm.at[pl.ds(off, EC)], es)
        pltpu.sync_copy(v_hbm.at[pl.ds(off, EC)], ev)

        def zc(i, _):
            dsg = pl.ds(i * L, L)
            cvoc[dsg] = izero16
            cseg[dsg] = izero16
            cval[dsg] = zero16
            return 0
        lax.fori_loop(0, (EC + L) // L, zc, 0)

        def comp(g, cnt_vec):
            dsg = pl.ds(g * L, L)
            bb = eb[dsg]
            nn = en[dsg]
            ss = es[dsg]
            vv = ev[dsg]
            seg = bb * N + nn
            owned = lax.shift_right_arithmetic(seg, SEG_SHIFT) == widv
            segloc = jnp.bitwise_and(seg, SPT - 1)
            bs = bb * S + ss
            oi = owned.astype(jnp.int32)
            cum = _prefix_sum(oi)
            pc = _lanesum(oi)
            pos = cnt_vec + cum - 1
            plsc.store_scatter(cvoc, [pos], bs, mask=owned)
            plsc.store_scatter(cseg, [pos], segloc, mask=owned)
            plsc.store_scatter(cval, [pos], vv, mask=owned)
            return cnt_vec + pc
        cnt_vec = lax.fori_loop(0, EC // L, comp, izero16)
        cnt = cnt_vec[0]

        def voc(i, _):
            dsg = pl.ds(i * L, L)
            cvoc[dsg] = plsc.load_gather(sub_v, [cvoc[dsg]])
            return 0
        lax.fori_loop(0, lax.shift_right_logical(cnt + (L - 1), 4), voc, 0)

        ng = lax.shift_right_logical(cnt + (GC - 1), 4)

        def gat(g, _):
            cp = pltpu.make_async_copy(
                emb_hbm.at[cvoc.at[pl.ds(g * GC, GC)]], rowbuf, sem0)
            cp.start()
            cp.wait()
            base = g * GC
            seg16 = cseg[pl.ds(base, L)]
            val16 = cval[pl.ds(base, L)]
            for j in range(GC):
                sj = seg16[j]
                vjv = jnp.broadcast_to(val16[j], (L,))

                def col(cb, _):
                    for u in range(8):
                        ds_ = pl.ds(cb * 128 + u * 16, L)
                        plsc.addupdate(acc.at[sj, ds_], rowbuf[j, ds_] * vjv)
                    return 0
                lax.fori_loop(0, 6, col, 0)
            return 0
        lax.fori_loop(0, ng, gat, 0)
        return 0
    lax.fori_loop(0, NCHUNK, chunk, 0)

    inv_h = jnp.float32(1.0 / H)
    magicv = jnp.full((L,), MAGIC, jnp.int32)
    c15 = jnp.full((L,), 1.5, jnp.float32)
    ch = jnp.full((L,), 0.5, jnp.float32)

    def ln_row(r, _):
        def s1(cb, sv):
            for u in range(8):
                sv = sv + acc[r, pl.ds(cb * 128 + u * 16, L)]
            return sv
        sv = lax.fori_loop(0, 6, s1, zero16)
        meanv = _lanesum(sv) * inv_h

        def s2(cb, qv):
            for u in range(8):
                d = acc[r, pl.ds(cb * 128 + u * 16, L)] - meanv
                qv = qv + d * d
            return qv
        qv = lax.fori_loop(0, 6, s2, zero16)
        xv = _lanesum(qv) * inv_h + EPS
        iv = magicv - lax.shift_right_arithmetic(plsc.bitcast(xv, jnp.int32), 1)
        yv = plsc.bitcast(iv, jnp.float32)
        yv = yv * (c15 - ch * xv * yv * yv)
        yv = yv * (c15 - ch * xv * yv * yv)
        yv = yv * (c15 - ch * xv * yv * yv)

        def s3(cb, _):
            for u in range(8):
                ds_ = pl.ds(cb * 128 + u * 16, L)
                acc[r, ds_] = ((acc[r, ds_] - meanv) * yv * lnw_v[ds_]
                               + lnb_v[ds_])
            return 0
        lax.fori_loop(0, 6, s3, 0)
        return 0
    lax.fori_loop(0, SPT, ln_row, 0)

    pltpu.sync_copy(acc, out_hbm.at[pl.ds(wid * SPT, SPT)])


def kernel(subword_ids, mask_indices, mask_values, emb, ln_weight, ln_bias):
    sub_flat = subword_ids.reshape(B * S).astype(jnp.int32)
    b_idx = mask_indices[0].astype(jnp.int32)
    n_idx = mask_indices[1].astype(jnp.int32)
    s_idx = mask_indices[2].astype(jnp.int32)

    mesh = plsc.VectorSubcoreMesh(core_axis_name="c", subcore_axis_name="s")
    f = pl.kernel(
        _body,
        out_type=jax.ShapeDtypeStruct((SEGS, H), jnp.float32),
        mesh=mesh,
        compiler_params=pltpu.CompilerParams(needs_layout_passes=False),
        scratch_types=[
            pltpu.VMEM((SPT, H), jnp.float32),
            pltpu.VMEM((B * S,), jnp.int32),
            pltpu.VMEM((EC,), jnp.int32),
            pltpu.VMEM((EC,), jnp.int32),
            pltpu.VMEM((EC,), jnp.int32),
            pltpu.VMEM((EC,), jnp.float32),
            pltpu.VMEM((EC + L,), jnp.int32),
            pltpu.VMEM((EC + L,), jnp.int32),
            pltpu.VMEM((EC + L,), jnp.float32),
            pltpu.VMEM((GC, H), jnp.float32),
            pltpu.VMEM((H,), jnp.float32),
            pltpu.VMEM((H,), jnp.float32),
            pltpu.SemaphoreType.DMA,
        ],
    )
    out = f(sub_flat, b_idx, n_idx, s_idx, mask_values.astype(jnp.float32),
            emb, ln_weight, ln_bias)
    return out.reshape(B, N, H)
